# baseline (speedup 1.0000x reference)
import dataclasses
import functools

import jax
import jax.numpy as jnp
from jax.experimental import pallas as pl
from jax.experimental.pallas import tpu as pltpu
from jax.experimental.pallas import tpu_sc as plsc

_N = 10000
_DEG = 16
_M = 1024
_CDEG = 64
_B = 4096
_F = 256

_GW = 128


def _sc_mesh():
    return plsc.VectorSubcoreMesh(core_axis_name="core", subcore_axis_name="subcore")


def _sc_params():
    cp = pltpu.CompilerParams()
    if "needs_layout_passes" in pltpu.CompilerParams.__dataclass_fields__:
        cp = dataclasses.replace(cp, needs_layout_passes=False)
    return cp


def _gather_pipeline(table_hbm, idx_hbm, out_hbm, n_idx, row_w):
    def body(i_vmem, o_vmem):
        pltpu.sync_copy(table_hbm.at[i_vmem.at[0]], o_vmem)

    pltpu.emit_pipeline(
        body,
        grid=(n_idx // _GW,),
        in_specs=[pl.BlockSpec((1, _GW), lambda i: (0, i))],
        out_specs=[pl.BlockSpec((_GW, row_w), lambda i: (i, 0))],
        core_axis_name=("core", "subcore"),
        dimension_semantics=(pltpu.PARALLEL,),
    )(idx_hbm, out_hbm)


def _sc_gather_feat_nbrs(feature, flow_adj_pad, idx_a, idx_b):
    na = idx_a.shape[1]
    nb = idx_b.shape[1]

    @functools.partial(
        pl.kernel,
        mesh=_sc_mesh(),
        out_type=[
            jax.ShapeDtypeStruct((na, _F // 2), jnp.int32),
            jax.ShapeDtypeStruct((nb, 128), jnp.int32),
        ],
    )
    def k(feat_hbm, fadj_hbm, ia_hbm, ib_hbm, r1_hbm, nb_hbm):
        _gather_pipeline(feat_hbm, ia_hbm, r1_hbm, na, _F // 2)
        _gather_pipeline(fadj_hbm, ib_hbm, nb_hbm, nb, 128)

    return k(feature, flow_adj_pad, idx_a, idx_b)


def _sc_gather_rows_vals(table, e_vec, idx):
    n = idx.shape[1]
    w = table.shape[1]

    @functools.partial(
        pl.kernel,
        mesh=_sc_mesh(),
        out_type=[
            jax.ShapeDtypeStruct((n, w), table.dtype),
            jax.ShapeDtypeStruct((n // _GW, _GW), jnp.float32),
        ],
        scratch_types=[pltpu.VMEM((_N,), jnp.float32)],
        compiler_params=_sc_params(),
    )
    def k(tab_hbm, e_hbm, i_hbm, o_hbm, v_hbm, e_vmem):
        pltpu.sync_copy(e_hbm, e_vmem)

        def body(i_vmem, o_vmem, v_vmem):
            pltpu.sync_copy(tab_hbm.at[i_vmem.at[0]], o_vmem)
            for j in range(_GW // 16):
                idx16 = i_vmem[0, pl.ds(16 * j, 16)]
                v_vmem[0, pl.ds(16 * j, 16)] = plsc.load_gather(e_vmem, [idx16])

        pltpu.emit_pipeline(
            body,
            grid=(n // _GW,),
            in_specs=[pl.BlockSpec((1, _GW), lambda i: (0, i))],
            out_specs=[
                pl.BlockSpec((_GW, w), lambda i: (i, 0)),
                pl.BlockSpec((1, _GW), lambda i: (i, 0)),
            ],
            core_axis_name=("core", "subcore"),
            dimension_semantics=(pltpu.PARALLEL,),
        )(i_hbm, o_hbm, v_hbm)

    return k(table, e_vec, idx)


def _sc_gather_uv(u_tab, v_tab, idx_u, idx_v):
    nu = idx_u.shape[1]
    nv = idx_v.shape[1]

    @functools.partial(
        pl.kernel,
        mesh=_sc_mesh(),
        out_type=[
            jax.ShapeDtypeStruct((nu, _F), jnp.float32),
            jax.ShapeDtypeStruct((nv, _F), jnp.float32),
        ],
    )
    def k(u_hbm, v_hbm, iu_hbm, iv_hbm, uu_hbm, vv_hbm):
        _gather_pipeline(u_hbm, iu_hbm, uu_hbm, nu, _F)
        _gather_pipeline(v_hbm, iv_hbm, vv_hbm, nv, _F)

    return k(u_tab, v_tab, idx_u, idx_v)


def _segsum_body(r_ref, o_ref):
    for j in range(8):
        o_ref[j : j + 1, :] = jnp.sum(
            r_ref[pl.ds(j * _CDEG, _CDEG), :].astype(jnp.float32),
            axis=0, keepdims=True,
        )


def _tc_segsum(r_char):
    return pl.pallas_call(
        _segsum_body,
        grid=(_M // 8,),
        in_specs=[pl.BlockSpec((8 * _CDEG, _F), lambda i: (i, 0))],
        out_specs=pl.BlockSpec((8, _F), lambda i: (i, 0)),
        out_shape=jax.ShapeDtypeStruct((_M, _F), jnp.float32),
    )(r_char)


def _weights_body(f_ref, w_ref, b_ref, wf_ref, a_ref,
                  e_ref, wtp_ref, wbp_ref, cf_ref):
    w = w_ref[...]
    b = b_ref[...]
    wft = wf_ref[:_F, :]
    wfb = wf_ref[_F:, :]
    wa = jnp.dot(w, a_ref[...], preferred_element_type=jnp.float32)
    e_ref[...] = jnp.dot(f_ref[...], wa, preferred_element_type=jnp.float32)
    wtp_ref[...] = jnp.dot(w, wft, preferred_element_type=jnp.float32)
    wbp_ref[...] = jnp.dot(w, wfb, preferred_element_type=jnp.float32)
    cf_ref[...] = jnp.dot(b, wft + wfb, preferred_element_type=jnp.float32)


def _tc_weights(feature, w, b2, wf, a):
    shapes = [
        jax.ShapeDtypeStruct((_N, 1), jnp.float32),
        jax.ShapeDtypeStruct((_F, _F), jnp.float32),
        jax.ShapeDtypeStruct((_F, _F), jnp.float32),
        jax.ShapeDtypeStruct((1, _F), jnp.float32),
    ]
    return pl.pallas_call(
        _weights_body,
        out_shape=shapes,
    )(feature, w, b2, wf, a)


def _prep_body(g_ref, w_ref, b_ref, wc_ref, u_ref, v_ref):
    b = b_ref[...]
    c = jnp.dot(g_ref[...], w_ref[...], preferred_element_type=jnp.float32) + 64.0 * b
    u_ref[...] = jnp.dot(c, wc_ref[:_F, :], preferred_element_type=jnp.float32)
    v_ref[...] = jnp.dot(c, wc_ref[_F:, :], preferred_element_type=jnp.float32)


def _tc_prep(g, w, b2, wc):
    shapes = [
        jax.ShapeDtypeStruct((_M, _F), jnp.float32),
        jax.ShapeDtypeStruct((_M, _F), jnp.float32),
    ]
    return pl.pallas_call(
        _prep_body,
        out_shape=shapes,
    )(g, w, b2, wc)


_BB = 256


def _flow_latent(r, vals, fid, wtp, wbp, cf):
    m = jnp.max(vals, axis=1, keepdims=True)
    p = jnp.exp(vals - m)
    attn = p / jnp.sum(p, axis=1, keepdims=True)
    xagg = jnp.sum(r * attn[:, :, None], axis=1)
    return jnp.tanh(
        jnp.dot(fid, wtp, preferred_element_type=jnp.float32)
        + jnp.dot(xagg, wbp, preferred_element_type=jnp.float32)
        + cf
    )


def _flowdec_body(r2a_ref, r2n_ref, va_ref, vn_ref, fa_ref, fn_ref,
                  uua_ref, uun_ref, vva_ref, vvn_ref,
                  wtp_ref, wbp_ref, cf_ref, o_ref):
    wtp = wtp_ref[...]
    wbp = wbp_ref[...]
    cf = cf_ref[...]
    fla = _flow_latent(r2a_ref[...].astype(jnp.float32), va_ref[...],
                       fa_ref[...].astype(jnp.float32), wtp, wbp, cf)
    fln = _flow_latent(r2n_ref[...].astype(jnp.float32), vn_ref[...],
                       fn_ref[...].astype(jnp.float32), wtp, wbp, cf)
    uua = uua_ref[...]
    uun = uun_ref[...]
    vva = vva_ref[...]
    vvn = vvn_ref[...]

    def score(fl, u, v, k):
        cl = jax.nn.sigmoid(u + v)
        s = jnp.sum(fl * cl, axis=1, keepdims=True)
        o_ref[:, k : k + 1] = jax.nn.sigmoid(s)

    score(fla, uua, vva, 0)
    score(fla, uun, vva, 1)
    score(fln, uun, vvn, 2)
    score(fln, uua, vvn, 3)


def _tc_flowdec(r2, vals, r1, uu, vv, wtp, wbp, cf):
    off = (_CDEG * _M) // _BB
    half = _B // _BB
    return pl.pallas_call(
        _flowdec_body,
        grid=(half,),
        in_specs=[
            pl.BlockSpec((_BB, _DEG, _F), lambda i: (i, 0, 0)),
            pl.BlockSpec((_BB, _DEG, _F), lambda i: (i + half, 0, 0)),
            pl.BlockSpec((_BB, _DEG), lambda i: (i, 0)),
            pl.BlockSpec((_BB, _DEG), lambda i: (i + half, 0)),
            pl.BlockSpec((_BB, _F), lambda i: (i + off, 0)),
            pl.BlockSpec((_BB, _F), lambda i: (i + off + half, 0)),
            pl.BlockSpec((_BB, _F), lambda i: (i, 0)),
            pl.BlockSpec((_BB, _F), lambda i: (i + half, 0)),
            pl.BlockSpec((_BB, _F), lambda i: (i, 0)),
            pl.BlockSpec((_BB, _F), lambda i: (i + half, 0)),
            pl.BlockSpec((_F, _F), lambda i: (0, 0)),
            pl.BlockSpec((_F, _F), lambda i: (0, 0)),
            pl.BlockSpec((1, _F), lambda i: (0, 0)),
        ],
        out_specs=pl.BlockSpec((_BB, 4), lambda i: (i, 0)),
        out_shape=jax.ShapeDtypeStruct((_B, 4), jnp.float32),
    )(r2, r2, vals, vals, r1, r1, uu, uu, vv, vv, wtp, wbp, cf)


def kernel(feature, flow_adj, flow_char_adj, item_id, category, PA_level,
           weight_emb, bias_emb, weight_character, a_attn, weight_flow):
    feature = feature.astype(jnp.float32)
    featb = jax.lax.bitcast_convert_type(
        feature.astype(jnp.bfloat16).reshape(_N, _F // 2, 2), jnp.int32
    )
    ids = item_id.T.reshape(-1).astype(jnp.int32)
    idx_a = jnp.concatenate(
        [flow_char_adj.reshape(-1).astype(jnp.int32), ids]
    ).reshape(1, -1)
    idx_b = ids.reshape(1, -1)

    fadj_pad = jnp.pad(flow_adj.astype(jnp.int32), ((0, 0), (0, 128 - _DEG)))
    r1i, nb = _sc_gather_feat_nbrs(featb, fadj_pad, idx_a, idx_b)
    r1 = jax.lax.bitcast_convert_type(r1i, jnp.bfloat16).reshape(-1, _F)
    e, wtp, wbp, cf = _tc_weights(
        feature, weight_emb, bias_emb.reshape(1, _F), weight_flow, a_attn
    )
    r2i, vals = _sc_gather_rows_vals(
        featb, e.reshape(_N), nb[:, :_DEG].reshape(1, -1)
    )
    r2 = jax.lax.bitcast_convert_type(r2i, jnp.bfloat16).reshape(-1, _F)

    g = _tc_segsum(r1[: _CDEG * _M])
    u_tab, v_tab = _tc_prep(g, weight_emb, bias_emb.reshape(1, _F), weight_character)

    idx_u = jnp.concatenate(
        [category[:, 0], category[:, 1]]
    ).astype(jnp.int32).reshape(1, -1)
    idx_v = jnp.concatenate(
        [PA_level[:, 0], PA_level[:, 1]]
    ).astype(jnp.int32).reshape(1, -1)
    uu, vv = _sc_gather_uv(u_tab, v_tab, idx_u, idx_v)

    p = _tc_flowdec(
        r2.reshape(2 * _B, _DEG, _F), vals.reshape(2 * _B, _DEG), r1,
        uu, vv, wtp, wbp, cf
    )
    return (p[:, 0], p[:, 1], p[:, 2], p[:, 3])


# speedup vs baseline: 5.7700x; 5.7700x over previous
import dataclasses
import functools

import jax
import jax.numpy as jnp
from jax.experimental import pallas as pl
from jax.experimental.pallas import tpu as pltpu
from jax.experimental.pallas import tpu_sc as plsc

_N = 10000
_DEG = 16
_M = 1024
_CDEG = 64
_B = 4096
_F = 256

_GW = 128


def _sc_mesh():
    return plsc.VectorSubcoreMesh(core_axis_name="core", subcore_axis_name="subcore")


def _sc_params():
    cp = pltpu.CompilerParams()
    if "needs_layout_passes" in pltpu.CompilerParams.__dataclass_fields__:
        cp = dataclasses.replace(cp, needs_layout_passes=False)
    return cp


def _gather_pipeline(table_hbm, idx_hbm, out_hbm, n_idx, row_w):
    def body(i_vmem, o_vmem):
        pltpu.sync_copy(table_hbm.at[i_vmem.at[0]], o_vmem)

    pltpu.emit_pipeline(
        body,
        grid=(n_idx // _GW,),
        in_specs=[pl.BlockSpec((1, _GW), lambda i: (0, i))],
        out_specs=[pl.BlockSpec((_GW, row_w), lambda i: (i, 0))],
        core_axis_name=("core", "subcore"),
        dimension_semantics=(pltpu.PARALLEL,),
    )(idx_hbm, out_hbm)


def _sc_gather_feat_nbrs(feature, flow_adj, idx_a, idx_b):
    na = idx_a.shape[1]
    nb = idx_b.shape[1]

    @functools.partial(
        pl.kernel,
        mesh=_sc_mesh(),
        out_type=[
            jax.ShapeDtypeStruct((na, _F // 2), jnp.int32),
            jax.ShapeDtypeStruct((nb, _DEG), jnp.int32),
        ],
        compiler_params=dataclasses.replace(
            _sc_params(), use_tc_tiling_on_sc=False
        ),
    )
    def k(feat_hbm, fadj_hbm, ia_hbm, ib_hbm, r1_hbm, nb_hbm):
        _gather_pipeline(feat_hbm, ia_hbm, r1_hbm, na, _F // 2)
        _gather_pipeline(fadj_hbm, ib_hbm, nb_hbm, nb, _DEG)

    return k(feature, flow_adj, idx_a, idx_b)


def _sc_gather_rows_vals(table, e_vec, idx):
    n = idx.shape[1]
    w = table.shape[1]

    @functools.partial(
        pl.kernel,
        mesh=_sc_mesh(),
        out_type=[
            jax.ShapeDtypeStruct((n, w), table.dtype),
            jax.ShapeDtypeStruct((n // _GW, _GW), jnp.float32),
        ],
        scratch_types=[pltpu.VMEM((_N,), jnp.float32)],
        compiler_params=_sc_params(),
    )
    def k(tab_hbm, e_hbm, i_hbm, o_hbm, v_hbm, e_vmem):
        pltpu.sync_copy(e_hbm, e_vmem)

        def body(i_vmem, o_vmem, v_vmem):
            pltpu.sync_copy(tab_hbm.at[i_vmem.at[0]], o_vmem)
            for j in range(_GW // 16):
                idx16 = i_vmem[0, pl.ds(16 * j, 16)]
                v_vmem[0, pl.ds(16 * j, 16)] = plsc.load_gather(e_vmem, [idx16])

        pltpu.emit_pipeline(
            body,
            grid=(n // _GW,),
            in_specs=[pl.BlockSpec((1, _GW), lambda i: (0, i))],
            out_specs=[
                pl.BlockSpec((_GW, w), lambda i: (i, 0)),
                pl.BlockSpec((1, _GW), lambda i: (i, 0)),
            ],
            core_axis_name=("core", "subcore"),
            dimension_semantics=(pltpu.PARALLEL,),
        )(i_hbm, o_hbm, v_hbm)

    return k(table, e_vec, idx)


def _sc_gather_uv(u_tab, v_tab, idx_u, idx_v):
    nu = idx_u.shape[1]
    nv = idx_v.shape[1]

    @functools.partial(
        pl.kernel,
        mesh=_sc_mesh(),
        out_type=[
            jax.ShapeDtypeStruct((nu, _F), jnp.float32),
            jax.ShapeDtypeStruct((nv, _F), jnp.float32),
        ],
    )
    def k(u_hbm, v_hbm, iu_hbm, iv_hbm, uu_hbm, vv_hbm):
        _gather_pipeline(u_hbm, iu_hbm, uu_hbm, nu, _F)
        _gather_pipeline(v_hbm, iv_hbm, vv_hbm, nv, _F)

    return k(u_tab, v_tab, idx_u, idx_v)


_PB = 1000


def _pack_body(f_ref, o_ref):
    h = _F // 2
    fb = f_ref[...].astype(jnp.bfloat16)
    x = jnp.stack([fb[:, :h], fb[:, h:]], axis=1)
    o_ref[...] = pltpu.bitcast(x, jnp.int32).reshape(_PB, h)


def _tc_pack(feature):
    return pl.pallas_call(
        _pack_body,
        grid=(_N // _PB,),
        in_specs=[pl.BlockSpec((_PB, _F), lambda i: (i, 0))],
        out_specs=pl.BlockSpec((_PB, _F // 2), lambda i: (i, 0)),
        out_shape=jax.ShapeDtypeStruct((_N, _F // 2), jnp.int32),
    )(feature)


def _segsum_body(r_ref, o_ref):
    for j in range(8):
        seg = pltpu.bitcast(r_ref[pl.ds(j * _CDEG, _CDEG), :], jnp.bfloat16)
        seg = seg.reshape(_CDEG, 2, _F // 2).astype(jnp.float32)
        o_ref[j] = jnp.sum(seg, axis=0)


def _tc_segsum(r_char_i32):
    return pl.pallas_call(
        _segsum_body,
        grid=(_M // 8,),
        in_specs=[pl.BlockSpec((8 * _CDEG, _F // 2), lambda i: (i, 0))],
        out_specs=pl.BlockSpec((8, 2, _F // 2), lambda i: (i, 0, 0)),
        out_shape=jax.ShapeDtypeStruct((_M, 2, _F // 2), jnp.float32),
    )(r_char_i32)


def _weights_body(f_ref, w_ref, b_ref, wf_ref, a_ref,
                  e_ref, wtp_ref, wbp_ref, cf_ref):
    w = w_ref[...]
    b = b_ref[...]
    wft = wf_ref[:_F, :]
    wfb = wf_ref[_F:, :]
    wa = jnp.dot(w, a_ref[...], preferred_element_type=jnp.float32)
    e_ref[...] = jnp.dot(f_ref[...], wa, preferred_element_type=jnp.float32)
    wtp_ref[...] = jnp.dot(w, wft, preferred_element_type=jnp.float32)
    wbp_ref[...] = jnp.dot(w, wfb, preferred_element_type=jnp.float32)
    cf_ref[...] = jnp.dot(b, wft + wfb, preferred_element_type=jnp.float32)


def _tc_weights(feature, w, b2, wf, a):
    shapes = [
        jax.ShapeDtypeStruct((_N, 1), jnp.float32),
        jax.ShapeDtypeStruct((_F, _F), jnp.float32),
        jax.ShapeDtypeStruct((_F, _F), jnp.float32),
        jax.ShapeDtypeStruct((1, _F), jnp.float32),
    ]
    return pl.pallas_call(
        _weights_body,
        out_shape=shapes,
    )(feature, w, b2, wf, a)


def _prep_body(g_ref, w_ref, b_ref, wc_ref, u_ref, v_ref):
    b = b_ref[...]
    h = _F // 2
    c = (
        jnp.dot(g_ref[:, 0, :], w_ref[:h, :], preferred_element_type=jnp.float32)
        + jnp.dot(g_ref[:, 1, :], w_ref[h:, :], preferred_element_type=jnp.float32)
        + 64.0 * b
    )
    u_ref[...] = jnp.dot(c, wc_ref[:_F, :], preferred_element_type=jnp.float32)
    v_ref[...] = jnp.dot(c, wc_ref[_F:, :], preferred_element_type=jnp.float32)


def _tc_prep(g, w, b2, wc):
    shapes = [
        jax.ShapeDtypeStruct((_M, _F), jnp.float32),
        jax.ShapeDtypeStruct((_M, _F), jnp.float32),
    ]
    return pl.pallas_call(
        _prep_body,
        out_shape=shapes,
    )(g, w, b2, wc)


_BB = 256


def _flow_latent(r_i32, vals, fid_i32, wtp, wbp, cf):
    h = _F // 2
    r = pltpu.bitcast(r_i32, jnp.bfloat16)
    r = r.reshape(_BB, _DEG, 2, h).astype(jnp.float32)
    fid = pltpu.bitcast(fid_i32, jnp.bfloat16).astype(jnp.float32)
    m = jnp.max(vals, axis=1, keepdims=True)
    p = jnp.exp(vals - m)
    attn = p / jnp.sum(p, axis=1, keepdims=True)
    xagg = jnp.sum(r * attn[:, :, None, None], axis=1)
    return jnp.tanh(
        jnp.dot(fid[:, 0, :], wtp[:h, :], preferred_element_type=jnp.float32)
        + jnp.dot(fid[:, 1, :], wtp[h:, :], preferred_element_type=jnp.float32)
        + jnp.dot(xagg[:, 0, :], wbp[:h, :], preferred_element_type=jnp.float32)
        + jnp.dot(xagg[:, 1, :], wbp[h:, :], preferred_element_type=jnp.float32)
        + cf
    )


def _flowdec_body(r2a_ref, r2n_ref, va_ref, vn_ref, fa_ref, fn_ref,
                  uua_ref, uun_ref, vva_ref, vvn_ref,
                  wtp_ref, wbp_ref, cf_ref, o_ref):
    wtp = wtp_ref[...]
    wbp = wbp_ref[...]
    cf = cf_ref[...]
    fla = _flow_latent(r2a_ref[...], va_ref[...], fa_ref[...], wtp, wbp, cf)
    fln = _flow_latent(r2n_ref[...], vn_ref[...], fn_ref[...], wtp, wbp, cf)
    uua = uua_ref[...]
    uun = uun_ref[...]
    vva = vva_ref[...]
    vvn = vvn_ref[...]

    def score(fl, u, v, k):
        cl = jax.nn.sigmoid(u + v)
        s = jnp.sum(fl * cl, axis=1, keepdims=True)
        o_ref[:, k : k + 1] = jax.nn.sigmoid(s)

    score(fla, uua, vva, 0)
    score(fla, uun, vva, 1)
    score(fln, uun, vvn, 2)
    score(fln, uua, vvn, 3)


def _tc_flowdec(r2, vals, r1, uu, vv, wtp, wbp, cf):
    off = (_CDEG * _M) // _BB
    half = _B // _BB
    return pl.pallas_call(
        _flowdec_body,
        grid=(half,),
        in_specs=[
            pl.BlockSpec((_BB, _DEG, _F // 2), lambda i: (i, 0, 0)),
            pl.BlockSpec((_BB, _DEG, _F // 2), lambda i: (i + half, 0, 0)),
            pl.BlockSpec((_BB, _DEG), lambda i: (i, 0)),
            pl.BlockSpec((_BB, _DEG), lambda i: (i + half, 0)),
            pl.BlockSpec((_BB, 1, _F // 2), lambda i: (i + off, 0, 0)),
            pl.BlockSpec((_BB, 1, _F // 2), lambda i: (i + off + half, 0, 0)),
            pl.BlockSpec((_BB, _F), lambda i: (i, 0)),
            pl.BlockSpec((_BB, _F), lambda i: (i + half, 0)),
            pl.BlockSpec((_BB, _F), lambda i: (i, 0)),
            pl.BlockSpec((_BB, _F), lambda i: (i + half, 0)),
            pl.BlockSpec((_F, _F), lambda i: (0, 0)),
            pl.BlockSpec((_F, _F), lambda i: (0, 0)),
            pl.BlockSpec((1, _F), lambda i: (0, 0)),
        ],
        out_specs=pl.BlockSpec((_BB, 4), lambda i: (i, 0)),
        out_shape=jax.ShapeDtypeStruct((_B, 4), jnp.float32),
    )(r2, r2, vals, vals, r1, r1, uu, uu, vv, vv, wtp, wbp, cf)


def kernel(feature, flow_adj, flow_char_adj, item_id, category, PA_level,
           weight_emb, bias_emb, weight_character, a_attn, weight_flow):
    feature = feature.astype(jnp.float32)
    featb = _tc_pack(feature)
    ids = item_id.T.reshape(-1).astype(jnp.int32)
    idx_a = jnp.concatenate(
        [flow_char_adj.reshape(-1).astype(jnp.int32), ids]
    ).reshape(1, -1)
    idx_b = ids.reshape(1, -1)

    r1i, nb = _sc_gather_feat_nbrs(featb, flow_adj.astype(jnp.int32), idx_a, idx_b)
    e, wtp, wbp, cf = _tc_weights(
        feature, weight_emb, bias_emb.reshape(1, _F), weight_flow, a_attn
    )
    r2i, vals = _sc_gather_rows_vals(featb, e.reshape(_N), nb.reshape(1, -1))

    g = _tc_segsum(r1i[: _CDEG * _M])
    u_tab, v_tab = _tc_prep(g, weight_emb, bias_emb.reshape(1, _F), weight_character)

    idx_u = jnp.concatenate(
        [category[:, 0], category[:, 1]]
    ).astype(jnp.int32).reshape(1, -1)
    idx_v = jnp.concatenate(
        [PA_level[:, 0], PA_level[:, 1]]
    ).astype(jnp.int32).reshape(1, -1)
    uu, vv = _sc_gather_uv(u_tab, v_tab, idx_u, idx_v)

    p = _tc_flowdec(
        r2i.reshape(2 * _B, _DEG, _F // 2), vals.reshape(2 * _B, _DEG),
        r1i.reshape(-1, 1, _F // 2), uu, vv, wtp, wbp, cf
    )
    return (p[:, 0], p[:, 1], p[:, 2], p[:, 3])
